# Initial kernel scaffold; baseline (speedup 1.0000x reference)
import jax
import jax.numpy as jnp
from jax.experimental import pallas as pl


def kernel(hop_dist, hop_emb):
    raise NotImplementedError("write your pallas kernel here")


# speedup vs baseline: 16.5757x; 16.5757x over previous
import jax
import jax.numpy as jnp
from jax.experimental import pallas as pl


def _tc_body(dist_ref, embT_ref, out_ref):
    d = dist_ref[0]
    K = embT_ref.shape[1]
    CH = d.shape[1]
    iota = jax.lax.broadcasted_iota(jnp.int32, (K, CH), 0)
    oh = (iota == d).astype(jnp.float32)
    out_ref[0] = jnp.dot(embT_ref[...], oh, preferred_element_type=jnp.float32)


def kernel(hop_dist, hop_emb):
    B, L, _ = hop_dist.shape
    K, H = hop_emb.shape
    N = L * L
    CH = 8192
    dist_flat = hop_dist.reshape(B, 1, N)
    embT = hop_emb.T

    out = pl.pallas_call(
        _tc_body,
        grid=(B, N // CH),
        in_specs=[
            pl.BlockSpec((1, 1, CH), lambda b, c: (b, 0, c)),
            pl.BlockSpec((H, K), lambda b, c: (0, 0)),
        ],
        out_specs=pl.BlockSpec((1, H, CH), lambda b, c: (b, 0, c)),
        out_shape=jax.ShapeDtypeStruct((B, H, N), jnp.float32),
    )(dist_flat, embT)
    return out.reshape(B, H, L, L)
